# Initial kernel scaffold; baseline (speedup 1.0000x reference)
import jax
import jax.numpy as jnp
from jax.experimental import pallas as pl


def kernel(ids, tables, W, b):
    raise NotImplementedError("write your pallas kernel here")


# speedup vs baseline: 19.2408x; 19.2408x over previous
import functools

import jax
import jax.numpy as jnp
from jax import lax
from jax.experimental import pallas as pl
from jax.experimental.pallas import tpu as pltpu
from jax.experimental.pallas import tpu_sc as plsc

F = 26
V = 100000
D = 128
B = 4096
NC = 2
NS = 16
NW = NC * NS
CB = B // NW


def _sc_gather(tab_flat, ids):
  mesh = plsc.VectorSubcoreMesh(
      core_axis_name="c", subcore_axis_name="s", num_cores=NC, num_subcores=NS)

  @functools.partial(
      pl.kernel,
      out_type=jax.ShapeDtypeStruct((F * B, D), jnp.float32),
      mesh=mesh,
      scratch_types=[
          pltpu.VMEM((2, CB), jnp.int32),
          pltpu.VMEM((2, CB, D), jnp.float32),
          pltpu.SemaphoreType.DMA,
          pltpu.SemaphoreType.DMA,
          pltpu.SemaphoreType.DMA,
          pltpu.SemaphoreType.DMA,
      ],
  )
  def gather_k(tab_hbm, ids_hbm, out_hbm, idx_v, rows_v, g0, g1, o0, o1):
    w = lax.axis_index("s") * NC + lax.axis_index("c")
    col = w * CB
    gsem = (g0, g1)
    osem = (o0, o1)
    gcopies = [None, None]
    ocopies = [None, None]
    for f in range(F):
      slot = f % 2
      if ocopies[slot] is not None:
        ocopies[slot].wait()
      pltpu.sync_copy(ids_hbm.at[f, pl.ds(col, CB)], idx_v.at[slot])
      for j in range(CB // 16):
        sl = pl.ds(j * 16, 16)
        idx_v[slot, sl] = idx_v[slot, sl] + f * V
      gcopies[slot] = pltpu.async_copy(
          tab_hbm.at[idx_v.at[slot]], rows_v.at[slot], gsem[slot])
      prev = 1 - slot
      if f >= 1:
        gcopies[prev].wait()
        ocopies[prev] = pltpu.async_copy(
            rows_v.at[prev],
            out_hbm.at[pl.ds((f - 1) * B + col, CB)],
            osem[prev])
    last = (F - 1) % 2
    gcopies[last].wait()
    ocopies[last] = pltpu.async_copy(
        rows_v.at[last],
        out_hbm.at[pl.ds((F - 1) * B + col, CB)],
        osem[last])
    ocopies[1 - last].wait()
    ocopies[last].wait()

  return gather_k(tab_flat, ids)


def _mm_body(e_ref, w_ref, b_ref, o_ref):
  f = pl.program_id(0)
  part = lax.dot_general(
      e_ref[0], w_ref[...],
      (((1,), (1,)), ((), ())),
      preferred_element_type=jnp.float32)

  @pl.when(f == 0)
  def _():
    o_ref[...] = part

  @pl.when(f > 0)
  def _():
    o_ref[...] = o_ref[...] + part

  @pl.when(f == F - 1)
  def _():
    o_ref[...] = jnp.maximum(o_ref[...] + b_ref[...], 0.0)


def _tc_matmul(e3, W, b2):
  return pl.pallas_call(
      _mm_body,
      grid=(F,),
      in_specs=[
          pl.BlockSpec((1, B, D), lambda f: (f, 0, 0)),
          pl.BlockSpec((D, D), lambda f: (0, f)),
          pl.BlockSpec((1, D), lambda f: (0, 0)),
      ],
      out_specs=pl.BlockSpec((B, D), lambda f: (0, 0)),
      out_shape=jax.ShapeDtypeStruct((B, D), jnp.float32),
      compiler_params=pltpu.CompilerParams(
          dimension_semantics=("arbitrary",)),
  )(e3, W, b2)


def kernel(ids, tables, W, b):
  tab_flat = tables.reshape(F * V, D)
  e = _sc_gather(tab_flat, ids)
  e3 = e.reshape(F, B, D)
  return _tc_matmul(e3, W, b.reshape(1, D))


# speedup vs baseline: 20.8216x; 1.0822x over previous
import functools

import jax
import jax.numpy as jnp
from jax import lax
from jax.experimental import pallas as pl
from jax.experimental.pallas import tpu as pltpu
from jax.experimental.pallas import tpu_sc as plsc

F = 26
V = 100000
D = 128
B = 4096
NC = 2
NS = 16
NW = NC * NS
CB = B // NW


NSLOT = 4


def _sc_gather(tab_flat, ids):
  mesh = plsc.VectorSubcoreMesh(
      core_axis_name="c", subcore_axis_name="s", num_cores=NC, num_subcores=NS)

  @functools.partial(
      pl.kernel,
      out_type=jax.ShapeDtypeStruct((F * B, D), jnp.float32),
      mesh=mesh,
      scratch_types=[
          pltpu.VMEM((F, CB), jnp.int32),
          pltpu.VMEM((NSLOT, CB, D), jnp.float32),
          [pltpu.SemaphoreType.DMA] * NSLOT,
          [pltpu.SemaphoreType.DMA] * NSLOT,
      ],
  )
  def gather_k(tab_hbm, ids_hbm, out_hbm, idx_v, rows_v, gsem, osem):
    w = lax.axis_index("s") * NC + lax.axis_index("c")
    col = w * CB
    pltpu.sync_copy(ids_hbm.at[:, pl.ds(col, CB)], idx_v)
    for f in range(1, F):
      for j in range(CB // 16):
        sl = pl.ds(j * 16, 16)
        idx_v[f, sl] = idx_v[f, sl] + f * V
    gcopies = [None] * F
    ocopies = [None] * F

    def start_out(f):
      gcopies[f].wait()
      ocopies[f] = pltpu.async_copy(
          rows_v.at[f % NSLOT],
          out_hbm.at[pl.ds(f * B + col, CB)],
          osem[f % NSLOT])

    for f in range(F):
      slot = f % NSLOT
      if f >= NSLOT:
        ocopies[f - NSLOT].wait()
      gcopies[f] = pltpu.async_copy(
          tab_hbm.at[idx_v.at[f]], rows_v.at[slot], gsem[slot])
      if f >= NSLOT - 1:
        start_out(f - (NSLOT - 1))
    for f in range(F - NSLOT + 1, F):
      start_out(f)
    for f in range(F - NSLOT, F):
      ocopies[f].wait()

  return gather_k(tab_flat, ids)


def _mm_body(e_ref, w_ref, b_ref, o_ref):
  f = pl.program_id(0)
  part = lax.dot_general(
      e_ref[0], w_ref[...],
      (((1,), (1,)), ((), ())),
      preferred_element_type=jnp.float32)

  @pl.when(f == 0)
  def _():
    o_ref[...] = part

  @pl.when(f > 0)
  def _():
    o_ref[...] = o_ref[...] + part

  @pl.when(f == F - 1)
  def _():
    o_ref[...] = jnp.maximum(o_ref[...] + b_ref[...], 0.0)


def _tc_matmul(e3, W, b2):
  return pl.pallas_call(
      _mm_body,
      grid=(F,),
      in_specs=[
          pl.BlockSpec((1, B, D), lambda f: (f, 0, 0)),
          pl.BlockSpec((D, D), lambda f: (0, f)),
          pl.BlockSpec((1, D), lambda f: (0, 0)),
      ],
      out_specs=pl.BlockSpec((B, D), lambda f: (0, 0)),
      out_shape=jax.ShapeDtypeStruct((B, D), jnp.float32),
      compiler_params=pltpu.CompilerParams(
          dimension_semantics=("arbitrary",)),
  )(e3, W, b2)


def kernel(ids, tables, W, b):
  tab_flat = tables.reshape(F * V, D)
  e = _sc_gather(tab_flat, ids)
  e3 = e.reshape(F, B, D)
  return _tc_matmul(e3, W, b.reshape(1, D))


# speedup vs baseline: 21.2639x; 1.0212x over previous
import functools

import jax
import jax.numpy as jnp
from jax import lax
from jax.experimental import pallas as pl
from jax.experimental.pallas import tpu as pltpu
from jax.experimental.pallas import tpu_sc as plsc

F = 26
V = 100000
D = 128
B = 4096
NC = 2
NS = 16
NW = NC * NS
CB = B // NW
NSLOT = 4
SPLITS = (13, 13)


def _sc_gather(tab_flat, ids_chunk, f_base, fc):
  mesh = plsc.VectorSubcoreMesh(
      core_axis_name="c", subcore_axis_name="s", num_cores=NC, num_subcores=NS)

  @functools.partial(
      pl.kernel,
      out_type=jax.ShapeDtypeStruct((fc * B, D), jnp.float32),
      mesh=mesh,
      scratch_types=[
          pltpu.VMEM((fc, CB), jnp.int32),
          pltpu.VMEM((NSLOT, CB, D), jnp.float32),
          [pltpu.SemaphoreType.DMA] * NSLOT,
          [pltpu.SemaphoreType.DMA] * NSLOT,
      ],
  )
  def gather_k(tab_hbm, ids_hbm, out_hbm, idx_v, rows_v, gsem, osem):
    w = lax.axis_index("s") * NC + lax.axis_index("c")
    col = w * CB
    pltpu.sync_copy(ids_hbm.at[:, pl.ds(col, CB)], idx_v)
    for f in range(fc):
      off = (f_base + f) * V
      if off == 0:
        continue
      for j in range(CB // 16):
        sl = pl.ds(j * 16, 16)
        idx_v[f, sl] = idx_v[f, sl] + off
    gcopies = [None] * fc
    ocopies = [None] * fc

    def start_out(f):
      gcopies[f].wait()
      ocopies[f] = pltpu.async_copy(
          rows_v.at[f % NSLOT],
          out_hbm.at[pl.ds(f * B + col, CB)],
          osem[f % NSLOT])

    for f in range(fc):
      slot = f % NSLOT
      if f >= NSLOT:
        ocopies[f - NSLOT].wait()
      gcopies[f] = pltpu.async_copy(
          tab_hbm.at[idx_v.at[f]], rows_v.at[slot], gsem[slot])
      if f >= NSLOT - 1:
        start_out(f - (NSLOT - 1))
    for f in range(max(fc - NSLOT + 1, 0), fc):
      start_out(f)
    for f in range(max(fc - NSLOT, 0), fc):
      ocopies[f].wait()

  return gather_k(tab_flat, ids_chunk)


def _mm_first_body(fc):
  def body(e_ref, w_ref, o_ref):
    f = pl.program_id(0)
    part = lax.dot_general(
        e_ref[0], w_ref[...],
        (((1,), (1,)), ((), ())),
        preferred_element_type=jnp.float32)

    @pl.when(f == 0)
    def _():
      o_ref[...] = part

    @pl.when(f > 0)
    def _():
      o_ref[...] = o_ref[...] + part

  return body


def _mm_last_body(fc):
  def body(e_ref, w_ref, b_ref, acc_ref, o_ref):
    f = pl.program_id(0)
    part = lax.dot_general(
        e_ref[0], w_ref[...],
        (((1,), (1,)), ((), ())),
        preferred_element_type=jnp.float32)

    @pl.when(f == 0)
    def _():
      o_ref[...] = acc_ref[...] + part

    @pl.when(f > 0)
    def _():
      o_ref[...] = o_ref[...] + part

    @pl.when(f == fc - 1)
    def _():
      o_ref[...] = jnp.maximum(o_ref[...] + b_ref[...], 0.0)

  return body


def _tc_matmul_first(e3, w_full, fc, f_base):
  return pl.pallas_call(
      _mm_first_body(fc),
      grid=(fc,),
      in_specs=[
          pl.BlockSpec((1, B, D), lambda f: (f, 0, 0)),
          pl.BlockSpec((D, D), lambda f, fb=f_base: (0, fb + f)),
      ],
      out_specs=pl.BlockSpec((B, D), lambda f: (0, 0)),
      out_shape=jax.ShapeDtypeStruct((B, D), jnp.float32),
      compiler_params=pltpu.CompilerParams(
          dimension_semantics=("arbitrary",)),
  )(e3, w_full)


def _tc_matmul_last(e3, w_full, b2, acc, fc, f_base):
  return pl.pallas_call(
      _mm_last_body(fc),
      grid=(fc,),
      in_specs=[
          pl.BlockSpec((1, B, D), lambda f: (f, 0, 0)),
          pl.BlockSpec((D, D), lambda f, fb=f_base: (0, fb + f)),
          pl.BlockSpec((1, D), lambda f: (0, 0)),
          pl.BlockSpec((B, D), lambda f: (0, 0)),
      ],
      out_specs=pl.BlockSpec((B, D), lambda f: (0, 0)),
      out_shape=jax.ShapeDtypeStruct((B, D), jnp.float32),
      compiler_params=pltpu.CompilerParams(
          dimension_semantics=("arbitrary",)),
  )(e3, w_full, b2, acc)


def kernel(ids, tables, W, b):
  tab_flat = tables.reshape(F * V, D)
  fc0, fc1 = SPLITS
  e0 = _sc_gather(tab_flat, ids[:fc0], 0, fc0)
  e1 = _sc_gather(tab_flat, ids[fc0:], fc0, fc1)
  acc = _tc_matmul_first(e0.reshape(fc0, B, D), W, fc0, 0)
  out = _tc_matmul_last(e1.reshape(fc1, B, D), W, b.reshape(1, D), acc, fc1, fc0)
  return out
